# baseline (speedup 1.0000x reference)
import functools

import jax
import jax.numpy as jnp
from jax import lax
from jax.experimental import pallas as pl
from jax.experimental.pallas import tpu as pltpu
from jax.experimental.pallas import tpu_sc as plsc

NC = 2
NS = 16
LANES = 16
W = 128


def _make_gin_agg(N, E, feature_split):
    K = 80
    NB = 25
    nworkers = NS if feature_split else NC * NS
    EPS = E // nworkers
    assert E % nworkers == 0 and EPS % (K * NB) == 0
    NCH = EPS // K
    NSTG = NCH // NB
    RPS = (N // NS) // 8 * 8
    TAIL = N - NS * RPS
    mesh = plsc.VectorSubcoreMesh(
        core_axis_name="c", subcore_axis_name="s",
        num_cores=NC, num_subcores=NS)

    def _copy_rows(src_ref, soff, dst_ref, doff, s):
        r0 = pl.multiple_of(s * RPS, 8)
        pltpu.sync_copy(src_ref.at[pl.ds(pl.multiple_of(soff + r0, 8), RPS)],
                        dst_ref.at[pl.ds(pl.multiple_of(doff + r0, 8), RPS)])
        if TAIL:
            @pl.when(s == NS - 1)
            def _tail():
                tr = NS * RPS
                pltpu.sync_copy(
                    src_ref.at[pl.ds(pl.multiple_of(soff + tr, 8), TAIL)],
                    dst_ref.at[pl.ds(pl.multiple_of(doff + tr, 8), TAIL)])

    scratch = [
        pltpu.VMEM_SHARED((N, W), jnp.float32),
        pltpu.VMEM((NB, K), jnp.int32),
        pltpu.VMEM((NB, K), jnp.int32),
        pltpu.VMEM((K, W), jnp.float32),
        pltpu.VMEM((K, W), jnp.float32),
        pltpu.SemaphoreType.DMA,
        pltpu.SemaphoreType.DMA,
        pltpu.SemaphoreType.DMA,
        pltpu.SemaphoreType.DMA,
    ]
    out_type = jax.ShapeDtypeStruct((2 * N, W), jnp.float32)

    def _edge_loop(table_hbm, src_at, dst_at, acc, srcb, dstb, rows,
                   gsems, ssems):
        def stage(t, carry):
            pltpu.sync_copy(src_at(t), srcb)
            pltpu.sync_copy(dst_at(t), dstb)
            pltpu.async_copy(table_hbm.at[srcb.at[0]], rows[0], gsems[0])

            def outer(g, c2):
                for b in range(2):
                    i = g * 2 + b
                    nb = 1 - b

                    @pl.when(i < NB)
                    def _proc():
                        @pl.when(i >= 1)
                        def _w_scat():
                            pltpu.make_async_copy(
                                rows[nb], acc.at[dstb.at[i - 1]],
                                ssems[nb]).wait()

                        @pl.when(i + 1 < NB)
                        def _pref():
                            pltpu.async_copy(table_hbm.at[srcb.at[i + 1]],
                                             rows[nb], gsems[nb])
                        pltpu.make_async_copy(table_hbm.at[srcb.at[i]],
                                              rows[b], gsems[b]).wait()
                        pltpu.async_copy(rows[b], acc.at[dstb.at[i]],
                                         ssems[b], add=True)
                return c2

            lax.fori_loop(0, (NB + 1) // 2, outer, 0)
            lb = (NB - 1) % 2
            pltpu.make_async_copy(rows[lb], acc.at[dstb.at[NB - 1]],
                                  ssems[lb]).wait()
            return carry

        lax.fori_loop(0, NSTG, stage, 0)

    if feature_split:
        @functools.partial(
            pl.kernel, out_type=out_type, mesh=mesh, scratch_types=scratch)
        def agg_kernel(hs_hbm, src5_hbm, dst4_hbm, zs_hbm, acc, srcb, dstb,
                       rows0, rows1, gsem0, gsem1, ssem0, ssem1):
            c = lax.axis_index("c")
            s = lax.axis_index("s")
            cN = c * N
            _copy_rows(hs_hbm, cN, acc, 0, s)
            plsc.subcore_barrier()
            _edge_loop(hs_hbm,
                       lambda t: src5_hbm.at[c, s, t],
                       lambda t: dst4_hbm.at[s, t],
                       acc, srcb, dstb, (rows0, rows1),
                       (gsem0, gsem1), (ssem0, ssem1))
            plsc.subcore_barrier()
            _copy_rows(acc, 0, zs_hbm, cN, s)
    else:
        @functools.partial(
            pl.kernel, out_type=out_type, mesh=mesh, scratch_types=scratch)
        def agg_kernel(h_hbm, zero_hbm, src4_hbm, dst4_hbm, zs_hbm, acc,
                       srcb, dstb, rows0, rows1, gsem0, gsem1, ssem0, ssem1):
            c = lax.axis_index("c")
            s = lax.axis_index("s")
            w = c * NS + s

            @pl.when(c == 0)
            def _seed_h():
                _copy_rows(h_hbm, 0, acc, 0, s)
            @pl.when(c == 1)
            def _seed_zero():
                _copy_rows(zero_hbm, 0, acc, 0, s)
            plsc.subcore_barrier()
            _edge_loop(h_hbm,
                       lambda t: src4_hbm.at[w, t],
                       lambda t: dst4_hbm.at[w, t],
                       acc, srcb, dstb, (rows0, rows1),
                       (gsem0, gsem1), (ssem0, ssem1))
            plsc.subcore_barrier()
            _copy_rows(acc, 0, zs_hbm, c * N, s)

    return agg_kernel


def _make_mlp(N, Din, combine, R=400):
    H = 256
    Wh = Din if combine == "add" else Din // 2
    nblk = N // R

    def body(z0_ref, z1_ref, wa_ref, wb_ref, out_ref):
        if combine == "add":
            z = z0_ref[...] + z1_ref[...]
        else:
            z = jnp.concatenate([z0_ref[...], z1_ref[...]], axis=1)
        t = jnp.dot(z, wa_ref[...], preferred_element_type=jnp.float32)
        t = jnp.maximum(t, 0.0)
        t = jnp.dot(t, wb_ref[...], preferred_element_type=jnp.float32)
        t = jnp.maximum(t, 0.0)
        out_ref[...] = jnp.stack([t[:, :128], t[:, 128:]], axis=0)

    return pl.pallas_call(
        body,
        grid=(nblk,),
        in_specs=[
            pl.BlockSpec((R, Wh), lambda i: (i, 0)),
            pl.BlockSpec((R, Wh), lambda i: (i + nblk, 0)),
            pl.BlockSpec((Din, H), lambda i: (0, 0)),
            pl.BlockSpec((H, H), lambda i: (0, 0)),
        ],
        out_specs=pl.BlockSpec((2, R, 128), lambda i: (0, i, 0)),
        out_shape=jax.ShapeDtypeStruct((2, N, 128), jnp.float32),
    )


def _make_head(N, G=128, H=256, R=400):
    nblk = N // R
    NEG = float("-inf")

    def body(h0_ref, h1_ref, b_ref, m1w_ref, m1b_ref, m2w_ref, m2b_ref,
             owt_ref, ob_ref, out_ref, sum_acc, max_acc):
        i = pl.program_id(0)

        @pl.when(i == 0)
        def _init():
            sum_acc[...] = jnp.zeros_like(sum_acc)
            max_acc[...] = jnp.full_like(max_acc, NEG)

        h = jnp.concatenate([h0_ref[...], h1_ref[...]], axis=1)
        brow = b_ref[...].reshape(1, R)
        seg = lax.broadcasted_iota(jnp.int32, (G, R), 0)
        mask = (seg == jnp.broadcast_to(brow, (G, R))).astype(jnp.float32)
        haug = jnp.concatenate(
            [h, jnp.ones((R, 128), jnp.float32)], axis=1)
        sum_acc[...] = sum_acc[...] + jnp.dot(
            mask, haug, preferred_element_type=jnp.float32)

        g_lo = b_ref[0, 0, 0]
        g_hi = b_ref[0, 0, R - 1]
        bcol = brow.reshape(R, 1)

        def upd(g, carry):
            vals = jnp.where(bcol == g, h, NEG)
            m = jnp.max(vals, axis=0, keepdims=True)
            cur = max_acc[pl.ds(g, 1), :]
            max_acc[pl.ds(g, 1), :] = jnp.maximum(cur, m)
            return carry

        lax.fori_loop(g_lo, g_hi + 1, upd, 0)

        @pl.when(i == nblk - 1)
        def _final():
            sums = sum_acc[...]
            cnt = jnp.max(sums[:, H:], axis=1, keepdims=True)
            sump = sums[:, :H]
            meanp = sump / jnp.maximum(cnt, 1.0)
            hid = jnp.concatenate([max_acc[...], meanp, sump], axis=1)
            t = jnp.dot(hid, m1w_ref[...], preferred_element_type=jnp.float32)
            t = jnp.maximum(t + m1b_ref[...], 0.0)
            t = jnp.dot(t, m2w_ref[...], preferred_element_type=jnp.float32)
            t = jnp.maximum(t + m2b_ref[...], 0.0)
            out = jnp.sum(t * owt_ref[...], axis=1, keepdims=True) + ob_ref[...]
            out_ref[...] = out

    return pl.pallas_call(
        body,
        grid=(nblk,),
        in_specs=[
            pl.BlockSpec((R, 128), lambda i: (i, 0)),
            pl.BlockSpec((R, 128), lambda i: (i + nblk, 0)),
            pl.BlockSpec((1, 1, R), lambda i: (i, 0, 0)),
            pl.BlockSpec((3 * H, 256), lambda i: (0, 0)),
            pl.BlockSpec((1, 256), lambda i: (0, 0)),
            pl.BlockSpec((256, 128), lambda i: (0, 0)),
            pl.BlockSpec((1, 128), lambda i: (0, 0)),
            pl.BlockSpec((1, 128), lambda i: (0, 0)),
            pl.BlockSpec((1, 1), lambda i: (0, 0)),
        ],
        out_specs=pl.BlockSpec((G, 1), lambda i: (0, 0)),
        out_shape=jax.ShapeDtypeStruct((G, 1), jnp.float32),
        scratch_shapes=[
            pltpu.VMEM((G, H + 128), jnp.float32),
            pltpu.VMEM((G, H), jnp.float32),
        ],
    )


def kernel(x, edge_index, batch_index, W1a, W1b, W2a, W2b, W3a, W3b,
           M1w, M1b, M2w, M2b, Ow, Ob):
    N, D = x.shape
    E = edge_index.shape[1]
    H = 256
    G = 128
    K = 80
    NB = 25
    src = edge_index[0]
    dst = edge_index[1]
    src_e = src.reshape(NC * NS, -1, NB, K)
    dst_e = dst.reshape(NC * NS, -1, NB, K)
    src_f = jnp.stack([src, src + N]).reshape(NC, NS, -1, NB, K)
    dst_f = dst.reshape(NS, -1, NB, K)
    zero = jnp.zeros((N, D), jnp.float32)

    agg1 = _make_gin_agg(N, E, feature_split=False)
    agg23 = _make_gin_agg(N, E, feature_split=True)
    mlp1 = _make_mlp(N, D, combine="add")
    mlp23 = _make_mlp(N, H, combine="cat")
    head = _make_head(N, G, H)

    z1 = agg1(x, zero, src_e, dst_e)
    h1 = mlp1(z1, z1, W1a, W1b).reshape(2 * N, 128)
    z2 = agg23(h1, src_f, dst_f)
    h2 = mlp23(z2, z2, W2a, W2b).reshape(2 * N, 128)
    z3 = agg23(h2, src_f, dst_f)
    h3 = mlp23(z3, z3, W3a, W3b).reshape(2 * N, 128)

    bi3 = batch_index.reshape(N // 400, 1, 400)
    out = head(h3, h3, bi3, M1w, M1b.reshape(1, -1), M2w, M2b.reshape(1, -1),
               Ow.reshape(1, -1), Ob.reshape(1, 1))
    return out


# speedup vs baseline: 1.1544x; 1.1544x over previous
import functools

import jax
import jax.numpy as jnp
from jax import lax
from jax.experimental import pallas as pl
from jax.experimental.pallas import tpu as pltpu
from jax.experimental.pallas import tpu_sc as plsc

NC = 2
NS = 16
LANES = 16
W = 128


def _make_gin_agg(N, E, feature_split):
    K = 80
    NB = 25
    nworkers = NS if feature_split else NC * NS
    EPS = E // nworkers
    assert E % nworkers == 0 and EPS % (K * NB) == 0
    NCH = EPS // K
    NSTG = NCH // NB
    RPS = (N // NS) // 8 * 8
    TAIL = N - NS * RPS
    mesh = plsc.VectorSubcoreMesh(
        core_axis_name="c", subcore_axis_name="s",
        num_cores=NC, num_subcores=NS)

    def _copy_rows(src_ref, soff, dst_ref, doff, s):
        r0 = pl.multiple_of(s * RPS, 8)
        pltpu.sync_copy(src_ref.at[pl.ds(pl.multiple_of(soff + r0, 8), RPS)],
                        dst_ref.at[pl.ds(pl.multiple_of(doff + r0, 8), RPS)])
        if TAIL:
            @pl.when(s == NS - 1)
            def _tail():
                tr = NS * RPS
                pltpu.sync_copy(
                    src_ref.at[pl.ds(pl.multiple_of(soff + tr, 8), TAIL)],
                    dst_ref.at[pl.ds(pl.multiple_of(doff + tr, 8), TAIL)])

    scratch = [
        pltpu.VMEM_SHARED((N, W), jnp.float32),
        pltpu.VMEM((NB, K), jnp.int32),
        pltpu.VMEM((NB, K), jnp.int32),
        pltpu.VMEM((K, W), jnp.float32),
        pltpu.VMEM((K, W), jnp.float32),
        pltpu.VMEM((K, W), jnp.float32),
        pltpu.SemaphoreType.DMA,
        pltpu.SemaphoreType.DMA,
        pltpu.SemaphoreType.DMA,
    ]
    out_type = jax.ShapeDtypeStruct((2 * N, W), jnp.float32)

    def _edge_loop(table_hbm, src_at, dst_at, acc, srcb, dstb, rows,
                   gsems, ssems):
        def stage(t, carry):
            pltpu.sync_copy(src_at(t), srcb)
            pltpu.sync_copy(dst_at(t), dstb)
            pltpu.async_copy(table_hbm.at[srcb.at[0]], rows[0], gsems[0])
            pltpu.async_copy(table_hbm.at[srcb.at[1]], rows[1], gsems[1])

            def outer(g, c2):
                for b in range(3):
                    i = g * 3 + b
                    nb = (b + 2) % 3

                    @pl.when(i < NB)
                    def _proc():
                        @pl.when(i + 2 < NB)
                        def _pref():
                            pltpu.async_copy(table_hbm.at[srcb.at[i + 2]],
                                             rows[nb], gsems[nb])
                        pltpu.make_async_copy(table_hbm.at[srcb.at[i]],
                                              rows[b], gsems[b]).wait()
                        pltpu.sync_copy(rows[b], acc.at[dstb.at[i]], add=True)
                return c2

            lax.fori_loop(0, (NB + 2) // 3, outer, 0)
            return carry

        lax.fori_loop(0, NSTG, stage, 0)

    if feature_split:
        @functools.partial(
            pl.kernel, out_type=out_type, mesh=mesh, scratch_types=scratch)
        def agg_kernel(hs_hbm, src5_hbm, dst4_hbm, zs_hbm, acc, srcb, dstb,
                       rows0, rows1, rows2, gsem0, gsem1, gsem2):
            c = lax.axis_index("c")
            s = lax.axis_index("s")
            cN = c * N
            _copy_rows(hs_hbm, cN, acc, 0, s)
            plsc.subcore_barrier()
            _edge_loop(hs_hbm,
                       lambda t: src5_hbm.at[c, s, t],
                       lambda t: dst4_hbm.at[s, t],
                       acc, srcb, dstb, (rows0, rows1, rows2),
                       (gsem0, gsem1, gsem2), ())
            plsc.subcore_barrier()
            _copy_rows(acc, 0, zs_hbm, cN, s)
    else:
        @functools.partial(
            pl.kernel, out_type=out_type, mesh=mesh, scratch_types=scratch)
        def agg_kernel(h_hbm, zero_hbm, src4_hbm, dst4_hbm, zs_hbm, acc,
                       srcb, dstb, rows0, rows1, rows2, gsem0, gsem1, gsem2):
            c = lax.axis_index("c")
            s = lax.axis_index("s")
            w = c * NS + s

            @pl.when(c == 0)
            def _seed_h():
                _copy_rows(h_hbm, 0, acc, 0, s)
            @pl.when(c == 1)
            def _seed_zero():
                _copy_rows(zero_hbm, 0, acc, 0, s)
            plsc.subcore_barrier()
            _edge_loop(h_hbm,
                       lambda t: src4_hbm.at[w, t],
                       lambda t: dst4_hbm.at[w, t],
                       acc, srcb, dstb, (rows0, rows1, rows2),
                       (gsem0, gsem1, gsem2), ())
            plsc.subcore_barrier()
            _copy_rows(acc, 0, zs_hbm, c * N, s)

    return agg_kernel


def _make_mlp(N, Din, combine, R=400):
    H = 256
    Wh = Din if combine == "add" else Din // 2
    nblk = N // R

    def body(z0_ref, z1_ref, wa_ref, wb_ref, out_ref):
        if combine == "add":
            z = z0_ref[...] + z1_ref[...]
        else:
            z = jnp.concatenate([z0_ref[...], z1_ref[...]], axis=1)
        t = jnp.dot(z, wa_ref[...], preferred_element_type=jnp.float32)
        t = jnp.maximum(t, 0.0)
        t = jnp.dot(t, wb_ref[...], preferred_element_type=jnp.float32)
        t = jnp.maximum(t, 0.0)
        out_ref[...] = jnp.stack([t[:, :128], t[:, 128:]], axis=0)

    return pl.pallas_call(
        body,
        grid=(nblk,),
        in_specs=[
            pl.BlockSpec((R, Wh), lambda i: (i, 0)),
            pl.BlockSpec((R, Wh), lambda i: (i + nblk, 0)),
            pl.BlockSpec((Din, H), lambda i: (0, 0)),
            pl.BlockSpec((H, H), lambda i: (0, 0)),
        ],
        out_specs=pl.BlockSpec((2, R, 128), lambda i: (0, i, 0)),
        out_shape=jax.ShapeDtypeStruct((2, N, 128), jnp.float32),
    )


def _make_head(N, G=128, H=256, R=400):
    nblk = N // R
    NEG = float("-inf")

    def body(h0_ref, h1_ref, b_ref, m1w_ref, m1b_ref, m2w_ref, m2b_ref,
             owt_ref, ob_ref, out_ref, sum_acc, max_acc):
        i = pl.program_id(0)

        @pl.when(i == 0)
        def _init():
            sum_acc[...] = jnp.zeros_like(sum_acc)
            max_acc[...] = jnp.full_like(max_acc, NEG)

        h = jnp.concatenate([h0_ref[...], h1_ref[...]], axis=1)
        brow = b_ref[...].reshape(1, R)
        seg = lax.broadcasted_iota(jnp.int32, (G, R), 0)
        mask = (seg == jnp.broadcast_to(brow, (G, R))).astype(jnp.float32)
        haug = jnp.concatenate(
            [h, jnp.ones((R, 128), jnp.float32)], axis=1)
        sum_acc[...] = sum_acc[...] + jnp.dot(
            mask, haug, preferred_element_type=jnp.float32)

        g_lo = b_ref[0, 0, 0]
        g_hi = b_ref[0, 0, R - 1]
        bcol = brow.reshape(R, 1)

        def upd(g, carry):
            vals = jnp.where(bcol == g, h, NEG)
            m = jnp.max(vals, axis=0, keepdims=True)
            cur = max_acc[pl.ds(g, 1), :]
            max_acc[pl.ds(g, 1), :] = jnp.maximum(cur, m)
            return carry

        lax.fori_loop(g_lo, g_hi + 1, upd, 0)

        @pl.when(i == nblk - 1)
        def _final():
            sums = sum_acc[...]
            cnt = jnp.max(sums[:, H:], axis=1, keepdims=True)
            sump = sums[:, :H]
            meanp = sump / jnp.maximum(cnt, 1.0)
            hid = jnp.concatenate([max_acc[...], meanp, sump], axis=1)
            t = jnp.dot(hid, m1w_ref[...], preferred_element_type=jnp.float32)
            t = jnp.maximum(t + m1b_ref[...], 0.0)
            t = jnp.dot(t, m2w_ref[...], preferred_element_type=jnp.float32)
            t = jnp.maximum(t + m2b_ref[...], 0.0)
            out = jnp.sum(t * owt_ref[...], axis=1, keepdims=True) + ob_ref[...]
            out_ref[...] = out

    return pl.pallas_call(
        body,
        grid=(nblk,),
        in_specs=[
            pl.BlockSpec((R, 128), lambda i: (i, 0)),
            pl.BlockSpec((R, 128), lambda i: (i + nblk, 0)),
            pl.BlockSpec((1, 1, R), lambda i: (i, 0, 0)),
            pl.BlockSpec((3 * H, 256), lambda i: (0, 0)),
            pl.BlockSpec((1, 256), lambda i: (0, 0)),
            pl.BlockSpec((256, 128), lambda i: (0, 0)),
            pl.BlockSpec((1, 128), lambda i: (0, 0)),
            pl.BlockSpec((1, 128), lambda i: (0, 0)),
            pl.BlockSpec((1, 1), lambda i: (0, 0)),
        ],
        out_specs=pl.BlockSpec((G, 1), lambda i: (0, 0)),
        out_shape=jax.ShapeDtypeStruct((G, 1), jnp.float32),
        scratch_shapes=[
            pltpu.VMEM((G, H + 128), jnp.float32),
            pltpu.VMEM((G, H), jnp.float32),
        ],
    )


def kernel(x, edge_index, batch_index, W1a, W1b, W2a, W2b, W3a, W3b,
           M1w, M1b, M2w, M2b, Ow, Ob):
    N, D = x.shape
    E = edge_index.shape[1]
    H = 256
    G = 128
    K = 80
    NB = 25
    src = edge_index[0]
    dst = edge_index[1]
    src_e = src.reshape(NC * NS, -1, NB, K)
    dst_e = dst.reshape(NC * NS, -1, NB, K)
    src_f = jnp.stack([src, src + N]).reshape(NC, NS, -1, NB, K)
    dst_f = dst.reshape(NS, -1, NB, K)
    zero = jnp.zeros((N, D), jnp.float32)

    agg1 = _make_gin_agg(N, E, feature_split=False)
    agg23 = _make_gin_agg(N, E, feature_split=True)
    mlp1 = _make_mlp(N, D, combine="add")
    mlp23 = _make_mlp(N, H, combine="cat")
    head = _make_head(N, G, H)

    z1 = agg1(x, zero, src_e, dst_e)
    h1 = mlp1(z1, z1, W1a, W1b).reshape(2 * N, 128)
    z2 = agg23(h1, src_f, dst_f)
    h2 = mlp23(z2, z2, W2a, W2b).reshape(2 * N, 128)
    z3 = agg23(h2, src_f, dst_f)
    h3 = mlp23(z3, z3, W3a, W3b).reshape(2 * N, 128)

    bi3 = batch_index.reshape(N // 400, 1, 400)
    out = head(h3, h3, bi3, M1w, M1b.reshape(1, -1), M2w, M2b.reshape(1, -1),
               Ow.reshape(1, -1), Ob.reshape(1, 1))
    return out


# speedup vs baseline: 1.1588x; 1.0039x over previous
import functools

import jax
import jax.numpy as jnp
from jax import lax
from jax.experimental import pallas as pl
from jax.experimental.pallas import tpu as pltpu
from jax.experimental.pallas import tpu_sc as plsc

NC = 2
NS = 16
LANES = 16
W = 128


def _make_gin_agg(N, E, feature_split):
    K = 80
    NB = 25
    nworkers = NS if feature_split else NC * NS
    EPS = E // nworkers
    assert E % nworkers == 0 and EPS % (K * NB) == 0
    NCH = EPS // K
    NSTG = NCH // NB
    RPS = (N // NS) // 8 * 8
    TAIL = N - NS * RPS
    mesh = plsc.VectorSubcoreMesh(
        core_axis_name="c", subcore_axis_name="s",
        num_cores=NC, num_subcores=NS)

    def _copy_rows(src_ref, soff, dst_ref, doff, s):
        r0 = pl.multiple_of(s * RPS, 8)
        pltpu.sync_copy(src_ref.at[pl.ds(pl.multiple_of(soff + r0, 8), RPS)],
                        dst_ref.at[pl.ds(pl.multiple_of(doff + r0, 8), RPS)])
        if TAIL:
            @pl.when(s == NS - 1)
            def _tail():
                tr = NS * RPS
                pltpu.sync_copy(
                    src_ref.at[pl.ds(pl.multiple_of(soff + tr, 8), TAIL)],
                    dst_ref.at[pl.ds(pl.multiple_of(doff + tr, 8), TAIL)])

    scratch = [
        pltpu.VMEM_SHARED((N, W), jnp.float32),
        pltpu.VMEM((NB, K), jnp.int32),
        pltpu.VMEM((NB, K), jnp.int32),
        pltpu.VMEM((K, W), jnp.float32),
        pltpu.VMEM((K, W), jnp.float32),
        pltpu.VMEM((K, W), jnp.float32),
        pltpu.VMEM((K, W), jnp.float32),
        pltpu.SemaphoreType.DMA,
        pltpu.SemaphoreType.DMA,
        pltpu.SemaphoreType.DMA,
        pltpu.SemaphoreType.DMA,
    ]
    out_type = jax.ShapeDtypeStruct((2 * N, W), jnp.float32)

    def _edge_loop(table_hbm, src_at, dst_at, acc, srcb, dstb, rows,
                   gsems, ssems):
        def stage(t, carry):
            pltpu.sync_copy(src_at(t), srcb)
            pltpu.sync_copy(dst_at(t), dstb)
            pltpu.async_copy(table_hbm.at[srcb.at[0]], rows[0], gsems[0])
            pltpu.async_copy(table_hbm.at[srcb.at[1]], rows[1], gsems[1])
            pltpu.async_copy(table_hbm.at[srcb.at[2]], rows[2], gsems[2])

            def outer(g, c2):
                for b in range(4):
                    i = g * 4 + b
                    nb = (b + 3) % 4

                    @pl.when(i < NB)
                    def _proc():
                        @pl.when(i + 3 < NB)
                        def _pref():
                            pltpu.async_copy(table_hbm.at[srcb.at[i + 3]],
                                             rows[nb], gsems[nb])
                        pltpu.make_async_copy(table_hbm.at[srcb.at[i]],
                                              rows[b], gsems[b]).wait()
                        pltpu.sync_copy(rows[b], acc.at[dstb.at[i]], add=True)
                return c2

            lax.fori_loop(0, (NB + 3) // 4, outer, 0)
            return carry

        lax.fori_loop(0, NSTG, stage, 0)

    if feature_split:
        @functools.partial(
            pl.kernel, out_type=out_type, mesh=mesh, scratch_types=scratch)
        def agg_kernel(hs_hbm, src5_hbm, dst4_hbm, zs_hbm, acc, srcb, dstb,
                       rows0, rows1, rows2, rows3, gsem0, gsem1, gsem2, gsem3):
            c = lax.axis_index("c")
            s = lax.axis_index("s")
            cN = c * N
            _copy_rows(hs_hbm, cN, acc, 0, s)
            plsc.subcore_barrier()
            _edge_loop(hs_hbm,
                       lambda t: src5_hbm.at[c, s, t],
                       lambda t: dst4_hbm.at[s, t],
                       acc, srcb, dstb, (rows0, rows1, rows2, rows3),
                       (gsem0, gsem1, gsem2, gsem3), ())
            plsc.subcore_barrier()
            _copy_rows(acc, 0, zs_hbm, cN, s)
    else:
        @functools.partial(
            pl.kernel, out_type=out_type, mesh=mesh, scratch_types=scratch)
        def agg_kernel(h_hbm, zero_hbm, src4_hbm, dst4_hbm, zs_hbm, acc,
                       srcb, dstb, rows0, rows1, rows2, rows3,
                       gsem0, gsem1, gsem2, gsem3):
            c = lax.axis_index("c")
            s = lax.axis_index("s")
            w = c * NS + s

            @pl.when(c == 0)
            def _seed_h():
                _copy_rows(h_hbm, 0, acc, 0, s)
            @pl.when(c == 1)
            def _seed_zero():
                _copy_rows(zero_hbm, 0, acc, 0, s)
            plsc.subcore_barrier()
            _edge_loop(h_hbm,
                       lambda t: src4_hbm.at[w, t],
                       lambda t: dst4_hbm.at[w, t],
                       acc, srcb, dstb, (rows0, rows1, rows2, rows3),
                       (gsem0, gsem1, gsem2, gsem3), ())
            plsc.subcore_barrier()
            _copy_rows(acc, 0, zs_hbm, c * N, s)

    return agg_kernel


def _make_mlp(N, Din, combine, R=400):
    H = 256
    Wh = Din if combine == "add" else Din // 2
    nblk = N // R

    def body(z0_ref, z1_ref, wa_ref, wb_ref, out_ref):
        if combine == "add":
            z = z0_ref[...] + z1_ref[...]
        else:
            z = jnp.concatenate([z0_ref[...], z1_ref[...]], axis=1)
        t = jnp.dot(z, wa_ref[...], preferred_element_type=jnp.float32)
        t = jnp.maximum(t, 0.0)
        t = jnp.dot(t, wb_ref[...], preferred_element_type=jnp.float32)
        t = jnp.maximum(t, 0.0)
        out_ref[...] = jnp.stack([t[:, :128], t[:, 128:]], axis=0)

    return pl.pallas_call(
        body,
        grid=(nblk,),
        in_specs=[
            pl.BlockSpec((R, Wh), lambda i: (i, 0)),
            pl.BlockSpec((R, Wh), lambda i: (i + nblk, 0)),
            pl.BlockSpec((Din, H), lambda i: (0, 0)),
            pl.BlockSpec((H, H), lambda i: (0, 0)),
        ],
        out_specs=pl.BlockSpec((2, R, 128), lambda i: (0, i, 0)),
        out_shape=jax.ShapeDtypeStruct((2, N, 128), jnp.float32),
    )


def _make_head(N, G=128, H=256, R=400):
    nblk = N // R
    NEG = float("-inf")

    def body(h0_ref, h1_ref, b_ref, m1w_ref, m1b_ref, m2w_ref, m2b_ref,
             owt_ref, ob_ref, out_ref, sum_acc, max_acc):
        i = pl.program_id(0)

        @pl.when(i == 0)
        def _init():
            sum_acc[...] = jnp.zeros_like(sum_acc)
            max_acc[...] = jnp.full_like(max_acc, NEG)

        h = jnp.concatenate([h0_ref[...], h1_ref[...]], axis=1)
        brow = b_ref[...].reshape(1, R)
        seg = lax.broadcasted_iota(jnp.int32, (G, R), 0)
        mask = (seg == jnp.broadcast_to(brow, (G, R))).astype(jnp.float32)
        haug = jnp.concatenate(
            [h, jnp.ones((R, 128), jnp.float32)], axis=1)
        sum_acc[...] = sum_acc[...] + jnp.dot(
            mask, haug, preferred_element_type=jnp.float32)

        g_lo = b_ref[0, 0, 0]
        g_hi = b_ref[0, 0, R - 1]
        bcol = brow.reshape(R, 1)

        def upd(g, carry):
            vals = jnp.where(bcol == g, h, NEG)
            m = jnp.max(vals, axis=0, keepdims=True)
            cur = max_acc[pl.ds(g, 1), :]
            max_acc[pl.ds(g, 1), :] = jnp.maximum(cur, m)
            return carry

        lax.fori_loop(g_lo, g_hi + 1, upd, 0)

        @pl.when(i == nblk - 1)
        def _final():
            sums = sum_acc[...]
            cnt = jnp.max(sums[:, H:], axis=1, keepdims=True)
            sump = sums[:, :H]
            meanp = sump / jnp.maximum(cnt, 1.0)
            hid = jnp.concatenate([max_acc[...], meanp, sump], axis=1)
            t = jnp.dot(hid, m1w_ref[...], preferred_element_type=jnp.float32)
            t = jnp.maximum(t + m1b_ref[...], 0.0)
            t = jnp.dot(t, m2w_ref[...], preferred_element_type=jnp.float32)
            t = jnp.maximum(t + m2b_ref[...], 0.0)
            out = jnp.sum(t * owt_ref[...], axis=1, keepdims=True) + ob_ref[...]
            out_ref[...] = out

    return pl.pallas_call(
        body,
        grid=(nblk,),
        in_specs=[
            pl.BlockSpec((R, 128), lambda i: (i, 0)),
            pl.BlockSpec((R, 128), lambda i: (i + nblk, 0)),
            pl.BlockSpec((1, 1, R), lambda i: (i, 0, 0)),
            pl.BlockSpec((3 * H, 256), lambda i: (0, 0)),
            pl.BlockSpec((1, 256), lambda i: (0, 0)),
            pl.BlockSpec((256, 128), lambda i: (0, 0)),
            pl.BlockSpec((1, 128), lambda i: (0, 0)),
            pl.BlockSpec((1, 128), lambda i: (0, 0)),
            pl.BlockSpec((1, 1), lambda i: (0, 0)),
        ],
        out_specs=pl.BlockSpec((G, 1), lambda i: (0, 0)),
        out_shape=jax.ShapeDtypeStruct((G, 1), jnp.float32),
        scratch_shapes=[
            pltpu.VMEM((G, H + 128), jnp.float32),
            pltpu.VMEM((G, H), jnp.float32),
        ],
    )


def kernel(x, edge_index, batch_index, W1a, W1b, W2a, W2b, W3a, W3b,
           M1w, M1b, M2w, M2b, Ow, Ob):
    N, D = x.shape
    E = edge_index.shape[1]
    H = 256
    G = 128
    K = 80
    NB = 25
    src = edge_index[0]
    dst = edge_index[1]
    src_e = src.reshape(NC * NS, -1, NB, K)
    dst_e = dst.reshape(NC * NS, -1, NB, K)
    src_f = jnp.stack([src, src + N]).reshape(NC, NS, -1, NB, K)
    dst_f = dst.reshape(NS, -1, NB, K)
    zero = jnp.zeros((N, D), jnp.float32)

    agg1 = _make_gin_agg(N, E, feature_split=False)
    agg23 = _make_gin_agg(N, E, feature_split=True)
    mlp1 = _make_mlp(N, D, combine="add")
    mlp23 = _make_mlp(N, H, combine="cat")
    head = _make_head(N, G, H)

    z1 = agg1(x, zero, src_e, dst_e)
    h1 = mlp1(z1, z1, W1a, W1b).reshape(2 * N, 128)
    z2 = agg23(h1, src_f, dst_f)
    h2 = mlp23(z2, z2, W2a, W2b).reshape(2 * N, 128)
    z3 = agg23(h2, src_f, dst_f)
    h3 = mlp23(z3, z3, W3a, W3b).reshape(2 * N, 128)

    bi3 = batch_index.reshape(N // 400, 1, 400)
    out = head(h3, h3, bi3, M1w, M1b.reshape(1, -1), M2w, M2b.reshape(1, -1),
               Ow.reshape(1, -1), Ob.reshape(1, 1))
    return out


# speedup vs baseline: 1.2077x; 1.0422x over previous
import functools

import jax
import jax.numpy as jnp
from jax import lax
from jax.experimental import pallas as pl
from jax.experimental.pallas import tpu as pltpu
from jax.experimental.pallas import tpu_sc as plsc

NC = 2
NS = 16
LANES = 16
W = 128


def _make_gin_agg(N, E, feature_split):
    K = 80
    NB = 50 if feature_split else 25
    nworkers = NS if feature_split else NC * NS
    EPS = E // nworkers
    assert E % nworkers == 0 and EPS % (K * NB) == 0
    NCH = EPS // K
    NSTG = NCH // NB
    RPS = (N // NS) // 8 * 8
    TAIL = N - NS * RPS
    mesh = plsc.VectorSubcoreMesh(
        core_axis_name="c", subcore_axis_name="s",
        num_cores=NC, num_subcores=NS)

    def _copy_rows(src_ref, soff, dst_ref, doff, s):
        r0 = pl.multiple_of(s * RPS, 8)
        pltpu.sync_copy(src_ref.at[pl.ds(pl.multiple_of(soff + r0, 8), RPS)],
                        dst_ref.at[pl.ds(pl.multiple_of(doff + r0, 8), RPS)])
        if TAIL:
            @pl.when(s == NS - 1)
            def _tail():
                tr = NS * RPS
                pltpu.sync_copy(
                    src_ref.at[pl.ds(pl.multiple_of(soff + tr, 8), TAIL)],
                    dst_ref.at[pl.ds(pl.multiple_of(doff + tr, 8), TAIL)])

    DEPTH = 3 if feature_split else 4
    scratch = [
        pltpu.VMEM_SHARED((N, W), jnp.float32),
        pltpu.VMEM((NB, K), jnp.int32),
        pltpu.VMEM((NB, K), jnp.int32),
    ] + [pltpu.VMEM((K, W), jnp.float32)] * DEPTH \
      + [pltpu.SemaphoreType.DMA] * DEPTH
    out_type = jax.ShapeDtypeStruct((2 * N, W), jnp.float32)

    def _edge_loop(table_hbm, src_at, dst_at, acc, srcb, dstb, rows,
                   gsems, ssems):
        def stage(t, carry):
            pltpu.sync_copy(src_at(t), srcb)
            pltpu.sync_copy(dst_at(t), dstb)
            for d in range(DEPTH - 1):
                pltpu.async_copy(table_hbm.at[srcb.at[d]], rows[d], gsems[d])

            def outer(g, c2):
                for b in range(DEPTH):
                    i = g * DEPTH + b
                    nb = (b + DEPTH - 1) % DEPTH

                    @pl.when(i < NB)
                    def _proc():
                        @pl.when(i + DEPTH - 1 < NB)
                        def _pref():
                            pltpu.async_copy(
                                table_hbm.at[srcb.at[i + DEPTH - 1]],
                                rows[nb], gsems[nb])
                        pltpu.make_async_copy(table_hbm.at[srcb.at[i]],
                                              rows[b], gsems[b]).wait()
                        pltpu.sync_copy(rows[b], acc.at[dstb.at[i]], add=True)
                return c2

            lax.fori_loop(0, (NB + DEPTH - 1) // DEPTH, outer, 0)
            return carry

        lax.fori_loop(0, NSTG, stage, 0)

    if feature_split:
        @functools.partial(
            pl.kernel, out_type=out_type, mesh=mesh, scratch_types=scratch)
        def agg_kernel(hs_hbm, src5_hbm, dst4_hbm, zs_hbm, acc, srcb, dstb,
                       *bufs):
            rows, gsems = bufs[:DEPTH], bufs[DEPTH:]
            c = lax.axis_index("c")
            s = lax.axis_index("s")
            cN = c * N
            _copy_rows(hs_hbm, cN, acc, 0, s)
            plsc.subcore_barrier()
            _edge_loop(hs_hbm,
                       lambda t: src5_hbm.at[c, s, t],
                       lambda t: dst4_hbm.at[s, t],
                       acc, srcb, dstb, rows, gsems, ())
            plsc.subcore_barrier()
            _copy_rows(acc, 0, zs_hbm, cN, s)
    else:
        @functools.partial(
            pl.kernel, out_type=out_type, mesh=mesh, scratch_types=scratch)
        def agg_kernel(h_hbm, zero_hbm, src4_hbm, dst4_hbm, zs_hbm, acc,
                       srcb, dstb, *bufs):
            rows, gsems = bufs[:DEPTH], bufs[DEPTH:]
            c = lax.axis_index("c")
            s = lax.axis_index("s")
            w = c * NS + s

            @pl.when(c == 0)
            def _seed_h():
                _copy_rows(h_hbm, 0, acc, 0, s)
            @pl.when(c == 1)
            def _seed_zero():
                _copy_rows(zero_hbm, 0, acc, 0, s)
            plsc.subcore_barrier()
            _edge_loop(h_hbm,
                       lambda t: src4_hbm.at[w, t],
                       lambda t: dst4_hbm.at[w, t],
                       acc, srcb, dstb, rows, gsems, ())
            plsc.subcore_barrier()
            _copy_rows(acc, 0, zs_hbm, c * N, s)

    return agg_kernel


def _make_mlp(N, Din, combine, R=400):
    H = 256
    Wh = Din if combine == "add" else Din // 2
    nblk = N // R

    def body(z0_ref, z1_ref, wa_ref, wb_ref, out_ref):
        if combine == "add":
            z = z0_ref[...] + z1_ref[...]
        else:
            z = jnp.concatenate([z0_ref[...], z1_ref[...]], axis=1)
        t = jnp.dot(z, wa_ref[...], preferred_element_type=jnp.float32)
        t = jnp.maximum(t, 0.0)
        t = jnp.dot(t, wb_ref[...], preferred_element_type=jnp.float32)
        t = jnp.maximum(t, 0.0)
        out_ref[...] = jnp.stack([t[:, :128], t[:, 128:]], axis=0)

    return pl.pallas_call(
        body,
        grid=(nblk,),
        in_specs=[
            pl.BlockSpec((R, Wh), lambda i: (i, 0)),
            pl.BlockSpec((R, Wh), lambda i: (i + nblk, 0)),
            pl.BlockSpec((Din, H), lambda i: (0, 0)),
            pl.BlockSpec((H, H), lambda i: (0, 0)),
        ],
        out_specs=pl.BlockSpec((2, R, 128), lambda i: (0, i, 0)),
        out_shape=jax.ShapeDtypeStruct((2, N, 128), jnp.float32),
    )


def _make_head(N, G=128, H=256, R=400):
    nblk = N // R
    NEG = float("-inf")

    def body(h0_ref, h1_ref, b_ref, m1w_ref, m1b_ref, m2w_ref, m2b_ref,
             owt_ref, ob_ref, out_ref, sum_acc, max_acc):
        i = pl.program_id(0)

        @pl.when(i == 0)
        def _init():
            sum_acc[...] = jnp.zeros_like(sum_acc)
            max_acc[...] = jnp.full_like(max_acc, NEG)

        h = jnp.concatenate([h0_ref[...], h1_ref[...]], axis=1)
        brow = b_ref[...].reshape(1, R)
        seg = lax.broadcasted_iota(jnp.int32, (G, R), 0)
        mask = (seg == jnp.broadcast_to(brow, (G, R))).astype(jnp.float32)
        haug = jnp.concatenate(
            [h, jnp.ones((R, 128), jnp.float32)], axis=1)
        sum_acc[...] = sum_acc[...] + jnp.dot(
            mask, haug, preferred_element_type=jnp.float32)

        g_lo = b_ref[0, 0, 0]
        g_hi = b_ref[0, 0, R - 1]
        bcol = brow.reshape(R, 1)

        def upd(g, carry):
            vals = jnp.where(bcol == g, h, NEG)
            m = jnp.max(vals, axis=0, keepdims=True)
            cur = max_acc[pl.ds(g, 1), :]
            max_acc[pl.ds(g, 1), :] = jnp.maximum(cur, m)
            return carry

        lax.fori_loop(g_lo, g_hi + 1, upd, 0)

        @pl.when(i == nblk - 1)
        def _final():
            sums = sum_acc[...]
            cnt = jnp.max(sums[:, H:], axis=1, keepdims=True)
            sump = sums[:, :H]
            meanp = sump / jnp.maximum(cnt, 1.0)
            hid = jnp.concatenate([max_acc[...], meanp, sump], axis=1)
            t = jnp.dot(hid, m1w_ref[...], preferred_element_type=jnp.float32)
            t = jnp.maximum(t + m1b_ref[...], 0.0)
            t = jnp.dot(t, m2w_ref[...], preferred_element_type=jnp.float32)
            t = jnp.maximum(t + m2b_ref[...], 0.0)
            out = jnp.sum(t * owt_ref[...], axis=1, keepdims=True) + ob_ref[...]
            out_ref[...] = out

    return pl.pallas_call(
        body,
        grid=(nblk,),
        in_specs=[
            pl.BlockSpec((R, 128), lambda i: (i, 0)),
            pl.BlockSpec((R, 128), lambda i: (i + nblk, 0)),
            pl.BlockSpec((1, 1, R), lambda i: (i, 0, 0)),
            pl.BlockSpec((3 * H, 256), lambda i: (0, 0)),
            pl.BlockSpec((1, 256), lambda i: (0, 0)),
            pl.BlockSpec((256, 128), lambda i: (0, 0)),
            pl.BlockSpec((1, 128), lambda i: (0, 0)),
            pl.BlockSpec((1, 128), lambda i: (0, 0)),
            pl.BlockSpec((1, 1), lambda i: (0, 0)),
        ],
        out_specs=pl.BlockSpec((G, 1), lambda i: (0, 0)),
        out_shape=jax.ShapeDtypeStruct((G, 1), jnp.float32),
        scratch_shapes=[
            pltpu.VMEM((G, H + 128), jnp.float32),
            pltpu.VMEM((G, H), jnp.float32),
        ],
    )


def kernel(x, edge_index, batch_index, W1a, W1b, W2a, W2b, W3a, W3b,
           M1w, M1b, M2w, M2b, Ow, Ob):
    N, D = x.shape
    E = edge_index.shape[1]
    H = 256
    G = 128
    K = 80
    src = edge_index[0]
    dst = edge_index[1]
    src_e = src.reshape(NC * NS, -1, 25, K)
    dst_e = dst.reshape(NC * NS, -1, 25, K)
    src_f = jnp.stack([src, src + N]).reshape(NC, NS, -1, 50, K)
    dst_f = dst.reshape(NS, -1, 50, K)
    zero = jnp.zeros((N, D), jnp.float32)

    agg1 = _make_gin_agg(N, E, feature_split=False)
    agg23 = _make_gin_agg(N, E, feature_split=True)
    mlp1 = _make_mlp(N, D, combine="add")
    mlp23 = _make_mlp(N, H, combine="cat")
    head = _make_head(N, G, H)

    z1 = agg1(x, zero, src_e, dst_e)
    h1 = mlp1(z1, z1, W1a, W1b).reshape(2 * N, 128)
    z2 = agg23(h1, src_f, dst_f)
    h2 = mlp23(z2, z2, W2a, W2b).reshape(2 * N, 128)
    z3 = agg23(h2, src_f, dst_f)
    h3 = mlp23(z3, z3, W3a, W3b).reshape(2 * N, 128)

    bi3 = batch_index.reshape(N // 400, 1, 400)
    out = head(h3, h3, bi3, M1w, M1b.reshape(1, -1), M2w, M2b.reshape(1, -1),
               Ow.reshape(1, -1), Ob.reshape(1, 1))
    return out


# speedup vs baseline: 1.2094x; 1.0014x over previous
import functools

import jax
import jax.numpy as jnp
from jax import lax
from jax.experimental import pallas as pl
from jax.experimental.pallas import tpu as pltpu
from jax.experimental.pallas import tpu_sc as plsc

NC = 2
NS = 16
LANES = 16
W = 128


def _make_gin_agg(N, E, feature_split):
    K = 80
    NB = 50 if feature_split else 25
    nworkers = NS if feature_split else NC * NS
    EPS = E // nworkers
    assert E % nworkers == 0 and EPS % (K * NB) == 0
    NCH = EPS // K
    NSTG = NCH // NB
    RPS = (N // NS) // 8 * 8
    TAIL = N - NS * RPS
    mesh = plsc.VectorSubcoreMesh(
        core_axis_name="c", subcore_axis_name="s",
        num_cores=NC, num_subcores=NS)

    def _copy_rows(src_ref, soff, dst_ref, doff, s):
        r0 = pl.multiple_of(s * RPS, 8)
        pltpu.sync_copy(src_ref.at[pl.ds(pl.multiple_of(soff + r0, 8), RPS)],
                        dst_ref.at[pl.ds(pl.multiple_of(doff + r0, 8), RPS)])
        if TAIL:
            @pl.when(s == NS - 1)
            def _tail():
                tr = NS * RPS
                pltpu.sync_copy(
                    src_ref.at[pl.ds(pl.multiple_of(soff + tr, 8), TAIL)],
                    dst_ref.at[pl.ds(pl.multiple_of(doff + tr, 8), TAIL)])

    DEPTH = 3 if feature_split else 4
    scratch = [
        pltpu.VMEM_SHARED((N, W), jnp.float32),
        pltpu.VMEM((NB, K), jnp.int32),
        pltpu.VMEM((NB, K), jnp.int32),
    ] + [pltpu.VMEM((K, W), jnp.float32)] * DEPTH \
      + [pltpu.SemaphoreType.DMA] * DEPTH
    out_type = jax.ShapeDtypeStruct((2 * N, W), jnp.float32)

    def _edge_loop(table_hbm, src_at, dst_at, acc, srcb, dstb, rows, gsems):
        def stage(t, carry):
            pltpu.sync_copy(src_at(t), srcb)
            pltpu.sync_copy(dst_at(t), dstb)
            for d in range(DEPTH - 1):
                pltpu.async_copy(table_hbm.at[srcb.at[d]], rows[d], gsems[d])

            def outer(g, c2):
                for b in range(DEPTH):
                    i = g * DEPTH + b
                    nb = (b + DEPTH - 1) % DEPTH

                    @pl.when(i < NB)
                    def _proc():
                        @pl.when(i + DEPTH - 1 < NB)
                        def _pref():
                            pltpu.async_copy(
                                table_hbm.at[srcb.at[i + DEPTH - 1]],
                                rows[nb], gsems[nb])
                        pltpu.make_async_copy(table_hbm.at[srcb.at[i]],
                                              rows[b], gsems[b]).wait()
                        pltpu.sync_copy(rows[b], acc.at[dstb.at[i]], add=True)
                return c2

            lax.fori_loop(0, (NB + DEPTH - 1) // DEPTH, outer, 0)
            return carry

        lax.fori_loop(0, NSTG, stage, 0)

    if feature_split:
        @functools.partial(
            pl.kernel, out_type=out_type, mesh=mesh, scratch_types=scratch)
        def agg_kernel(hs_hbm, src5_hbm, dst4_hbm, zs_hbm, acc, srcb, dstb,
                       *bufs):
            rows, gsems = bufs[:DEPTH], bufs[DEPTH:]
            c = lax.axis_index("c")
            s = lax.axis_index("s")
            cN = c * N
            _copy_rows(hs_hbm, cN, acc, 0, s)
            plsc.subcore_barrier()
            _edge_loop(hs_hbm,
                       lambda t: src5_hbm.at[c, s, t],
                       lambda t: dst4_hbm.at[s, t],
                       acc, srcb, dstb, rows, gsems)
            plsc.subcore_barrier()
            _copy_rows(acc, 0, zs_hbm, cN, s)
    else:
        @functools.partial(
            pl.kernel, out_type=out_type, mesh=mesh, scratch_types=scratch)
        def agg_kernel(h_hbm, zero_hbm, src4_hbm, dst4_hbm, zs_hbm, acc,
                       srcb, dstb, *bufs):
            rows, gsems = bufs[:DEPTH], bufs[DEPTH:]
            c = lax.axis_index("c")
            s = lax.axis_index("s")
            w = c * NS + s

            @pl.when(c == 0)
            def _seed_h():
                _copy_rows(h_hbm, 0, acc, 0, s)
            @pl.when(c == 1)
            def _seed_zero():
                _copy_rows(zero_hbm, 0, acc, 0, s)
            plsc.subcore_barrier()
            _edge_loop(h_hbm,
                       lambda t: src4_hbm.at[w, t],
                       lambda t: dst4_hbm.at[w, t],
                       acc, srcb, dstb, rows, gsems)
            plsc.subcore_barrier()
            _copy_rows(acc, 0, zs_hbm, c * N, s)

    return agg_kernel


def _make_mlp(N, Din, combine, R=400):
    H = 256
    Wh = Din if combine == "add" else Din // 2
    nblk = N // R

    def body(z0_ref, z1_ref, wa_ref, wb_ref, out_ref):
        if combine == "add":
            z = z0_ref[...] + z1_ref[...]
        else:
            z = jnp.concatenate([z0_ref[...], z1_ref[...]], axis=1)
        t = jnp.dot(z, wa_ref[...], preferred_element_type=jnp.float32)
        t = jnp.maximum(t, 0.0)
        t = jnp.dot(t, wb_ref[...], preferred_element_type=jnp.float32)
        t = jnp.maximum(t, 0.0)
        out_ref[...] = jnp.stack([t[:, :128], t[:, 128:]], axis=0)

    return pl.pallas_call(
        body,
        grid=(nblk,),
        in_specs=[
            pl.BlockSpec((R, Wh), lambda i: (i, 0)),
            pl.BlockSpec((R, Wh), lambda i: (i + nblk, 0)),
            pl.BlockSpec((Din, H), lambda i: (0, 0)),
            pl.BlockSpec((H, H), lambda i: (0, 0)),
        ],
        out_specs=pl.BlockSpec((2, R, 128), lambda i: (0, i, 0)),
        out_shape=jax.ShapeDtypeStruct((2, N, 128), jnp.float32),
    )


def _make_head(N, G=128, H=256, R=400):
    nblk = N // R
    NEG = float("-inf")

    def body(h0_ref, h1_ref, b_ref, m1w_ref, m1b_ref, m2w_ref, m2b_ref,
             owt_ref, ob_ref, out_ref, sum_acc, max_acc):
        i = pl.program_id(0)

        @pl.when(i == 0)
        def _init():
            sum_acc[...] = jnp.zeros_like(sum_acc)
            max_acc[...] = jnp.full_like(max_acc, NEG)

        h = jnp.concatenate([h0_ref[...], h1_ref[...]], axis=1)
        brow = b_ref[...].reshape(1, R)
        seg = lax.broadcasted_iota(jnp.int32, (G, R), 0)
        mask = (seg == jnp.broadcast_to(brow, (G, R))).astype(jnp.float32)
        haug = jnp.concatenate(
            [h, jnp.ones((R, 128), jnp.float32)], axis=1)
        sum_acc[...] = sum_acc[...] + jnp.dot(
            mask, haug, preferred_element_type=jnp.float32)

        g_lo = b_ref[0, 0, 0]
        g_hi = b_ref[0, 0, R - 1]
        bcol = brow.reshape(R, 1)

        def upd(g, carry):
            vals = jnp.where(bcol == g, h, NEG)
            m = jnp.max(vals, axis=0, keepdims=True)
            cur = max_acc[pl.ds(g, 1), :]
            max_acc[pl.ds(g, 1), :] = jnp.maximum(cur, m)
            return carry

        lax.fori_loop(g_lo, g_hi + 1, upd, 0)

        @pl.when(i == nblk - 1)
        def _final():
            sums = sum_acc[...]
            cnt = jnp.max(sums[:, H:], axis=1, keepdims=True)
            sump = sums[:, :H]
            meanp = sump / jnp.maximum(cnt, 1.0)
            hid = jnp.concatenate([max_acc[...], meanp, sump], axis=1)
            t = jnp.dot(hid, m1w_ref[...], preferred_element_type=jnp.float32)
            t = jnp.maximum(t + m1b_ref[...], 0.0)
            t = jnp.dot(t, m2w_ref[...], preferred_element_type=jnp.float32)
            t = jnp.maximum(t + m2b_ref[...], 0.0)
            out = jnp.sum(t * owt_ref[...], axis=1, keepdims=True) + ob_ref[...]
            out_ref[...] = out

    return pl.pallas_call(
        body,
        grid=(nblk,),
        in_specs=[
            pl.BlockSpec((R, 128), lambda i: (i, 0)),
            pl.BlockSpec((R, 128), lambda i: (i + nblk, 0)),
            pl.BlockSpec((1, 1, R), lambda i: (i, 0, 0)),
            pl.BlockSpec((3 * H, 256), lambda i: (0, 0)),
            pl.BlockSpec((1, 256), lambda i: (0, 0)),
            pl.BlockSpec((256, 128), lambda i: (0, 0)),
            pl.BlockSpec((1, 128), lambda i: (0, 0)),
            pl.BlockSpec((1, 128), lambda i: (0, 0)),
            pl.BlockSpec((1, 1), lambda i: (0, 0)),
        ],
        out_specs=pl.BlockSpec((G, 1), lambda i: (0, 0)),
        out_shape=jax.ShapeDtypeStruct((G, 1), jnp.float32),
        scratch_shapes=[
            pltpu.VMEM((G, H + 128), jnp.float32),
            pltpu.VMEM((G, H), jnp.float32),
        ],
    )


def kernel(x, edge_index, batch_index, W1a, W1b, W2a, W2b, W3a, W3b,
           M1w, M1b, M2w, M2b, Ow, Ob):
    N, D = x.shape
    E = edge_index.shape[1]
    H = 256
    G = 128
    K = 80
    src = edge_index[0]
    dst = edge_index[1]
    src_e = src.reshape(NC * NS, -1, 25, K)
    dst_e = dst.reshape(NC * NS, -1, 25, K)
    src_f = jnp.stack([src, src + N]).reshape(NC, NS, -1, 50, K)
    dst_f = dst.reshape(NS, -1, 50, K)
    zero = jnp.zeros((N, D), jnp.float32)

    agg1 = _make_gin_agg(N, E, feature_split=False)
    agg23 = _make_gin_agg(N, E, feature_split=True)
    mlp1 = _make_mlp(N, D, combine="add")
    mlp23 = _make_mlp(N, H, combine="cat")
    head = _make_head(N, G, H)

    z1 = agg1(x, zero, src_e, dst_e)
    h1 = mlp1(z1, z1, W1a, W1b).reshape(2 * N, 128)
    z2 = agg23(h1, src_f, dst_f)
    h2 = mlp23(z2, z2, W2a, W2b).reshape(2 * N, 128)
    z3 = agg23(h2, src_f, dst_f)
    h3 = mlp23(z3, z3, W3a, W3b).reshape(2 * N, 128)

    bi3 = batch_index.reshape(N // 400, 1, 400)
    out = head(h3, h3, bi3, M1w, M1b.reshape(1, -1), M2w, M2b.reshape(1, -1),
               Ow.reshape(1, -1), Ob.reshape(1, 1))
    return out
